# Initial kernel scaffold; baseline (speedup 1.0000x reference)
import jax
import jax.numpy as jnp
from jax.experimental import pallas as pl


def kernel(X, gradient, hessian):
    raise NotImplementedError("write your pallas kernel here")


# speedup vs baseline: 71.2322x; 71.2322x over previous
import functools

import jax
import jax.numpy as jnp
from jax import lax
from jax.experimental import pallas as pl
from jax.experimental.pallas import tpu as pltpu
from jax.experimental.pallas import tpu_sc as plsc

NC = 2
NS = 16
L = 16

MAX_BIN = 256


def _sc_kernel(N, F, CH):
    FH = F // NC
    FG = FH // L
    NCHT = -(-N // CH)
    n_iters = -(-NCHT // NS)
    HIST = FH * MAX_BIN
    ROWS_PER_TILE = (2 * FH) // NS
    assert N % CH == 0 and CH % L == 0 and CH % 8 == 0

    mesh = plsc.VectorSubcoreMesh(core_axis_name="c", subcore_axis_name="s")

    @functools.partial(
        pl.kernel,
        out_type=(
            jax.ShapeDtypeStruct((1, F, MAX_BIN), jnp.float32),
            jax.ShapeDtypeStruct((1, F, MAX_BIN), jnp.float32),
        ),
        mesh=mesh,
        compiler_params=pltpu.CompilerParams(needs_layout_passes=False),
        scratch_types=[
            pltpu.VMEM((CH, F), jnp.int32),
            pltpu.VMEM((CH,), jnp.float32),
            pltpu.VMEM((CH,), jnp.float32),
            pltpu.VMEM((HIST,), jnp.float32),
            pltpu.VMEM((HIST,), jnp.float32),
            pltpu.VMEM((NS, MAX_BIN), jnp.float32),
            pltpu.VMEM((MAX_BIN,), jnp.float32),
            pltpu.VMEM_SHARED((NS, 2, HIST), jnp.float32),
        ],
    )
    def k(x_hbm, g_hbm, h_hbm, gl_hbm, hl_hbm,
          xb, gb, hb, hg, hh, tmp16, row_out, shared):
        c = lax.axis_index("c")
        s = lax.axis_index("s")

        zeros16 = jnp.zeros((L,), jnp.float32)

        def zero_body(i, _):
            hg[pl.ds(i * L, L)] = zeros16
            hh[pl.ds(i * L, L)] = zeros16
            return 0

        lax.fori_loop(0, HIST // L, zero_body, 0)

        lane = lax.iota(jnp.int32, L)
        bases = [lane * MAX_BIN + fg * (L * MAX_BIN) for fg in range(FG)]

        fbase = c * FH
        NGRP = CH // L

        def chunk_body(ci, _):
            cid = ci * NS + s

            @pl.when(cid < NCHT)
            def _():
                i0 = cid * CH
                pltpu.sync_copy(x_hbm.at[pl.ds(i0, CH)], xb)
                pltpu.sync_copy(g_hbm.at[pl.ds(i0, CH)], gb)
                pltpu.sync_copy(h_hbm.at[pl.ds(i0, CH)], hb)

                def group_body(gidx, _):
                    row0 = gidx * L
                    gvec = gb[pl.ds(row0, L)]
                    hvec = hb[pl.ds(row0, L)]
                    for i in range(L):
                        gv = jnp.full((L,), gvec[i], jnp.float32)
                        hv = jnp.full((L,), hvec[i], jnp.float32)
                        for fg in range(FG):
                            idx = (xb[row0 + i, pl.ds(fbase + fg * L, L)]
                                   + bases[fg])
                            plsc.addupdate_scatter(hg, [idx], gv)
                            plsc.addupdate_scatter(hh, [idx], hv)
                    return 0

                lax.fori_loop(0, NGRP, group_body, 0)

            return 0

        lax.fori_loop(0, n_iters, chunk_body, 0)

        pltpu.sync_copy(hg, shared.at[s, 0])
        pltpu.sync_copy(hh, shared.at[s, 1])
        plsc.subcore_barrier()

        for rr in range(ROWS_PER_TILE):
            r = s * ROWS_PER_TILE + rr
            a = rr % 2
            lf = (s * ROWS_PER_TILE + rr) // 2
            lf = r // 2
            pltpu.sync_copy(
                shared.at[:, a, pl.ds(lf * MAX_BIN, MAX_BIN)], tmp16)
            carry = jnp.float32(0.0)
            for kk in range(MAX_BIN // L):
                v = tmp16[0, pl.ds(kk * L, L)]
                for t in range(1, NS):
                    v = v + tmp16[t, pl.ds(kk * L, L)]
                pv = plsc.cumsum(v) + jnp.full((L,), carry, jnp.float32)
                row_out[pl.ds(kk * L, L)] = pv
                carry = carry + jnp.sum(v)
            out_ref = gl_hbm if a == 0 else hl_hbm
            pltpu.sync_copy(row_out, out_ref.at[0, fbase + lf])

    return k


def kernel(X, gradient, hessian):
    N, F = X.shape
    CH = 400
    k = _sc_kernel(N, F, CH)
    gl, hl = k(X, gradient, hessian)
    return (gl, hl)


# speedup vs baseline: 143.3543x; 2.0125x over previous
import functools

import jax
import jax.numpy as jnp
from jax import lax
from jax.experimental import pallas as pl
from jax.experimental.pallas import tpu as pltpu
from jax.experimental.pallas import tpu_sc as plsc

NC = 2
NS = 16
NW = NC * NS
L = 16

MAX_BIN = 256


def _phase1(N, F, CH):
    FG = F // L
    NCHT = N // CH
    n_iters = -(-NCHT // NW)
    if n_iters % 2:
        n_iters += 1
    HIST = F * MAX_BIN
    NGRP = CH // L

    mesh = plsc.VectorSubcoreMesh(core_axis_name="c", subcore_axis_name="s")

    @functools.partial(
        pl.kernel,
        out_type=jax.ShapeDtypeStruct((NW, 2 * HIST), jnp.float32),
        mesh=mesh,
        compiler_params=pltpu.CompilerParams(needs_layout_passes=False),
        scratch_types=[
            pltpu.VMEM((CH, F), jnp.int32),
            pltpu.VMEM((CH, F), jnp.int32),
            pltpu.VMEM((CH,), jnp.float32),
            pltpu.VMEM((CH,), jnp.float32),
            pltpu.VMEM((CH,), jnp.float32),
            pltpu.VMEM((CH,), jnp.float32),
            pltpu.VMEM((HIST,), jnp.float32),
            pltpu.VMEM((HIST,), jnp.float32),
            pltpu.SemaphoreType.DMA,
            pltpu.SemaphoreType.DMA,
        ],
    )
    def k(x_hbm, g_hbm, h_hbm, scr_hbm,
          xb0, xb1, gb0, gb1, hb0, hb1, hg, hh, s0, s1):
        c = lax.axis_index("c")
        s = lax.axis_index("s")
        w = c * NS + s
        sems = (s0, s1)
        xbs, gbs, hbs = (xb0, xb1), (gb0, gb1), (hb0, hb1)

        zeros16 = jnp.zeros((L,), jnp.float32)

        def zero_body(i, _):
            hg[pl.ds(i * L, L)] = zeros16
            hh[pl.ds(i * L, L)] = zeros16
            return 0

        lax.fori_loop(0, HIST // L, zero_body, 0)

        lane = lax.iota(jnp.int32, L)
        bases = [lane * MAX_BIN + fg * (L * MAX_BIN) for fg in range(FG)]

        def copies(ci, b):
            i0 = (ci * NW + w) * CH
            return (
                pltpu.make_async_copy(x_hbm.at[pl.ds(i0, CH)], xbs[b],
                                      sems[b]),
                pltpu.make_async_copy(g_hbm.at[pl.ds(i0, CH)], gbs[b],
                                      sems[b]),
                pltpu.make_async_copy(h_hbm.at[pl.ds(i0, CH)], hbs[b],
                                      sems[b]),
            )

        def valid(ci):
            return ci * NW + w < NCHT

        def issue(ci, b):
            @pl.when(valid(ci))
            def _():
                for cp in copies(ci, b):
                    cp.start()

        def wait(ci, b):
            for cp in copies(ci, b):
                cp.wait()

        def compute(b):
            @plsc.parallel_loop(0, NGRP, unroll=2)
            def _(gidx):
                row0 = gidx * L
                gvec = gbs[b][pl.ds(row0, L)]
                hvec = hbs[b][pl.ds(row0, L)]
                for i in range(L):
                    gv = jnp.full((L,), gvec[i], jnp.float32)
                    hv = jnp.full((L,), hvec[i], jnp.float32)
                    for fg in range(FG):
                        idx = xbs[b][row0 + i, pl.ds(fg * L, L)] + bases[fg]
                        plsc.addupdate_scatter(hg, [idx], gv)
                        plsc.addupdate_scatter(hh, [idx], hv)

        issue(0, 0)
        issue(1, 1)

        def outer(j, _):
            for b in range(2):
                ci = j * 2 + b

                @pl.when(valid(ci))
                def _():
                    wait(ci, b)
                    compute(b)

                issue(ci + 2, b)
            return 0

        lax.fori_loop(0, n_iters // 2, outer, 0)

        pltpu.sync_copy(hg, scr_hbm.at[w, pl.ds(0, HIST)])
        pltpu.sync_copy(hh, scr_hbm.at[w, pl.ds(HIST, HIST)])

    return k


def _phase2(F):
    HIST = F * MAX_BIN
    ROWS_PER_ARR = F // NW

    mesh = plsc.VectorSubcoreMesh(core_axis_name="c", subcore_axis_name="s")

    @functools.partial(
        pl.kernel,
        out_type=(
            jax.ShapeDtypeStruct((1, F, MAX_BIN), jnp.float32),
            jax.ShapeDtypeStruct((1, F, MAX_BIN), jnp.float32),
        ),
        mesh=mesh,
        compiler_params=pltpu.CompilerParams(needs_layout_passes=False),
        scratch_types=[
            pltpu.VMEM((NW, MAX_BIN), jnp.float32),
            pltpu.VMEM((MAX_BIN,), jnp.float32),
        ],
    )
    def k(scr_hbm, gl_hbm, hl_hbm, acc, row_out):
        c = lax.axis_index("c")
        s = lax.axis_index("s")
        w = c * NS + s

        for a, out_ref in ((0, gl_hbm), (1, hl_hbm)):
            for rr in range(ROWS_PER_ARR):
                f = w * ROWS_PER_ARR + rr
                roff = a * HIST + f * MAX_BIN
                pltpu.sync_copy(scr_hbm.at[:, pl.ds(roff, MAX_BIN)], acc)
                carry = jnp.float32(0.0)
                for kk in range(MAX_BIN // L):
                    v = acc[0, pl.ds(kk * L, L)]
                    for t in range(1, NW):
                        v = v + acc[t, pl.ds(kk * L, L)]
                    pv = plsc.cumsum(v) + jnp.full((L,), carry, jnp.float32)
                    row_out[pl.ds(kk * L, L)] = pv
                    carry = carry + jnp.sum(v)
                pltpu.sync_copy(row_out, out_ref.at[0, f])

    return k


def kernel(X, gradient, hessian):
    N, F = X.shape
    CH = 320
    scr = _phase1(N, F, CH)(X, gradient, hessian)
    gl, hl = _phase2(F)(scr)
    return (gl, hl)
